# Initial kernel scaffold; baseline (speedup 1.0000x reference)
import jax
import jax.numpy as jnp
from jax.experimental import pallas as pl


def kernel(image, w1, b1, w2, b2, w3, b3, w4, b4, w5, b5, fc1_w, fc1_b, fc2_w, fc2_b):
    raise NotImplementedError("write your pallas kernel here")


# speedup vs baseline: 35.1927x; 35.1927x over previous
import jax
import jax.numpy as jnp
from jax.experimental import pallas as pl
from jax.experimental.pallas import tpu as pltpu

_BINS = 32


def _img_kernel(img_ref, resized_ref, counts_ref):
    x = img_ref[0]
    r = x[0]
    g = 0.5 * (x[1] + x[2])
    b = x[3]
    gray = 0.299 * r + 0.587 * g + 0.114 * b

    ri = jax.lax.broadcasted_iota(jnp.int32, (112, 448), 0)
    ci = jax.lax.broadcasted_iota(jnp.int32, (112, 448), 1)
    P = jnp.where(ci // 4 == ri, 0.25, 0.0).astype(jnp.float32)
    ri2 = jax.lax.broadcasted_iota(jnp.int32, (448, 112), 0)
    ci2 = jax.lax.broadcasted_iota(jnp.int32, (448, 112), 1)
    PT = jnp.where(ri2 // 4 == ci2, 0.25, 0.0).astype(jnp.float32)
    for ch_i, ch in enumerate((r, g, b)):
        t = jnp.dot(P, ch, preferred_element_type=jnp.float32)
        resized_ref[0, ch_i] = jnp.dot(t, PT, preferred_element_type=jnp.float32)

    pr = jnp.concatenate([gray[:1], gray, gray[-1:]], axis=0)
    p = jnp.concatenate([pr[:, :1], pr, pr[:, -1:]], axis=1)
    sv = p[:-2] + 2.0 * p[1:-1] + p[2:]
    gx = (sv[:, 2:] - sv[:, :-2]) * 0.125
    sh = p[:, :-2] + 2.0 * p[:, 1:-1] + p[:, 2:]
    gy = (sh[2:] - sh[:-2]) * 0.125
    mag = jnp.sqrt(gx * gx + gy * gy + 1e-6)

    iidx = jnp.clip((gray * _BINS).astype(jnp.int32), 0, _BINS - 1)
    gidx = jnp.clip((mag * _BINS).astype(jnp.int32), 0, _BINS - 1)

    lane = jax.lax.broadcasted_iota(jnp.int32, (1, _BINS), 1)
    rows = []
    for idx in (iidx, gidx):
        acc = [jnp.zeros((1, _BINS), jnp.float32) for _ in range(4)]
        for c in range(_BINS):
            m = (idx == c).astype(jnp.float32)
            oh = (lane == c).astype(jnp.float32)
            acc[0] = acc[0] + jnp.sum(m[:224, :224]) * oh
            acc[1] = acc[1] + jnp.sum(m[:224, 224:]) * oh
            acc[2] = acc[2] + jnp.sum(m[224:, :224]) * oh
            acc[3] = acc[3] + jnp.sum(m[224:, 224:]) * oh
        rows.extend(acc)
    counts_ref[0] = jnp.concatenate(rows, axis=0)


def _head_kernel(sem_ref, inten_ref, grad_ref, w1_ref, b1_ref, w2_ref, b2_ref,
                 out_ref):
    def l2n(v):
        n = jnp.sqrt(jnp.sum(v * v, axis=1, keepdims=True))
        return v / (n + 1e-7)

    s = jax.nn.relu(l2n(sem_ref[:]))
    i = jax.nn.relu(l2n(inten_ref[:]))
    g = jax.nn.relu(l2n(grad_ref[:]))
    h = jnp.concatenate([s, i, g], axis=1)
    h1 = jax.nn.relu(jnp.dot(h, w1_ref[:], preferred_element_type=jnp.float32)
                     + b1_ref[:])
    out_ref[:] = (jnp.dot(h1, w2_ref[:], preferred_element_type=jnp.float32)
                  + b2_ref[:])


def _conv(x, w, b, stride, pad):
    y = jax.lax.conv_general_dilated(x, w, (stride, stride),
                                     [(pad, pad), (pad, pad)],
                                     dimension_numbers=('NCHW', 'OIHW', 'NCHW'))
    return y + b[None, :, None, None]


def _maxpool3s2(x):
    return jax.lax.reduce_window(x, -jnp.inf, jax.lax.max, (1, 1, 3, 3),
                                 (1, 1, 2, 2), 'VALID')


def kernel(image, w1, b1, w2, b2, w3, b3, w4, b4, w5, b5, fc1_w, fc1_b,
           fc2_w, fc2_b):
    B = image.shape[0]

    resized, counts = pl.pallas_call(
        _img_kernel,
        grid=(B,),
        in_specs=[pl.BlockSpec((1, 4, 448, 448), lambda i: (i, 0, 0, 0))],
        out_specs=[
            pl.BlockSpec((1, 3, 112, 112), lambda i: (i, 0, 0, 0)),
            pl.BlockSpec((1, 8, _BINS), lambda i: (i, 0, 0)),
        ],
        out_shape=[
            jax.ShapeDtypeStruct((B, 3, 112, 112), jnp.float32),
            jax.ShapeDtypeStruct((B, 8, _BINS), jnp.float32),
        ],
        compiler_params=pltpu.CompilerParams(
            dimension_semantics=("parallel",)),
    )(image)

    x = _maxpool3s2(jax.nn.relu(_conv(resized, w1, b1, 4, 2)))
    x = _maxpool3s2(jax.nn.relu(_conv(x, w2, b2, 1, 2)))
    x = jax.nn.relu(_conv(x, w3, b3, 1, 1))
    x = jax.nn.relu(_conv(x, w4, b4, 1, 1))
    x = _maxpool3s2(jax.nn.relu(_conv(x, w5, b5, 1, 1)))
    f_sem = x.reshape(B, -1)

    h1 = counts.reshape(B, 2, 2, 2, _BINS) / (224.0 * 224.0)
    h0 = h1.sum(axis=(2, 3)) * 0.25
    out0 = jnp.broadcast_to(h0[:, :, :, None, None], (B, 2, _BINS, 4, 4))
    t = h1.transpose(0, 1, 4, 2, 3)
    out1 = jnp.repeat(jnp.repeat(t, 2, axis=3), 2, axis=4)
    feat = jnp.concatenate([out0, out1, jnp.zeros_like(out0)], axis=2)
    feat = feat.reshape(B, 2, 3 * _BINS * 16)
    f_inten = feat[:, 0]
    f_grad = feat[:, 1]

    out = pl.pallas_call(
        _head_kernel,
        out_shape=jax.ShapeDtypeStruct((B, fc2_w.shape[0]), jnp.float32),
    )(f_sem, f_inten, f_grad, fc1_w.T, fc1_b.reshape(1, -1), fc2_w.T,
      fc2_b.reshape(1, -1))
    return out


# speedup vs baseline: 69.2722x; 1.9684x over previous
import jax
import jax.numpy as jnp
from jax.experimental import pallas as pl
from jax.experimental.pallas import tpu as pltpu

_BINS = 32


def _img_kernel(img_ref, resized_ref, counts_ref):
    x = img_ref[0]
    r = x[0]
    g = 0.5 * (x[1] + x[2])
    b = x[3]
    gray = 0.299 * r + 0.587 * g + 0.114 * b

    ri = jax.lax.broadcasted_iota(jnp.int32, (112, 448), 0)
    ci = jax.lax.broadcasted_iota(jnp.int32, (112, 448), 1)
    P = jnp.where(ci // 4 == ri, 0.25, 0.0).astype(jnp.float32)
    ri2 = jax.lax.broadcasted_iota(jnp.int32, (448, 112), 0)
    ci2 = jax.lax.broadcasted_iota(jnp.int32, (448, 112), 1)
    PT = jnp.where(ri2 // 4 == ci2, 0.25, 0.0).astype(jnp.float32)
    for ch_i, ch in enumerate((r, g, b)):
        t = jnp.dot(P, ch, preferred_element_type=jnp.float32)
        resized_ref[0, ch_i] = jnp.dot(t, PT, preferred_element_type=jnp.float32)

    pr = jnp.concatenate([gray[:1], gray, gray[-1:]], axis=0)
    p = jnp.concatenate([pr[:, :1], pr, pr[:, -1:]], axis=1)
    sv = p[:-2] + 2.0 * p[1:-1] + p[2:]
    gx = (sv[:, 2:] - sv[:, :-2]) * 0.125
    sh = p[:, :-2] + 2.0 * p[:, 1:-1] + p[:, 2:]
    gy = (sh[2:] - sh[:-2]) * 0.125
    mag = jnp.sqrt(gx * gx + gy * gy + 1e-6)

    iidx = jnp.clip((gray * _BINS).astype(jnp.int32), 0, _BINS - 1)
    gidx = jnp.clip((mag * _BINS).astype(jnp.int32), 0, _BINS - 1)

    lane = jax.lax.broadcasted_iota(jnp.int32, (1, _BINS), 1)
    rows = []
    for idx in (iidx, gidx):
        acc = [jnp.zeros((1, _BINS), jnp.float32) for _ in range(4)]
        for c in range(_BINS):
            oh = (lane == c).astype(jnp.float32)
            accT = jnp.zeros((8, 448), jnp.float32)
            accB = jnp.zeros((8, 448), jnp.float32)
            for i in range(28):
                accT = accT + (idx[8 * i:8 * i + 8] == c).astype(jnp.float32)
                accB = accB + (idx[224 + 8 * i:232 + 8 * i] == c).astype(
                    jnp.float32)
            acc[0] = acc[0] + jnp.sum(accT[:, :224]) * oh
            acc[1] = acc[1] + jnp.sum(accT[:, 224:]) * oh
            acc[2] = acc[2] + jnp.sum(accB[:, :224]) * oh
            acc[3] = acc[3] + jnp.sum(accB[:, 224:]) * oh
        rows.extend(acc)
    counts_ref[0] = jnp.concatenate(rows, axis=0)


def _head_kernel(sem_ref, inten_ref, grad_ref, w1_ref, b1_ref, w2_ref, b2_ref,
                 out_ref):
    def l2n(v):
        n = jnp.sqrt(jnp.sum(v * v, axis=1, keepdims=True))
        return v / (n + 1e-7)

    s = jax.nn.relu(l2n(sem_ref[:]))
    i = jax.nn.relu(l2n(inten_ref[:]))
    g = jax.nn.relu(l2n(grad_ref[:]))
    h = jnp.concatenate([s, i, g], axis=1)
    h1 = jax.nn.relu(jnp.dot(h, w1_ref[:], preferred_element_type=jnp.float32)
                     + b1_ref[:])
    out_ref[:] = (jnp.dot(h1, w2_ref[:], preferred_element_type=jnp.float32)
                  + b2_ref[:])


def _conv(x, w, b, stride, pad):
    y = jax.lax.conv_general_dilated(x, w, (stride, stride),
                                     [(pad, pad), (pad, pad)],
                                     dimension_numbers=('NCHW', 'OIHW', 'NCHW'))
    return y + b[None, :, None, None]


def _maxpool3s2(x):
    return jax.lax.reduce_window(x, -jnp.inf, jax.lax.max, (1, 1, 3, 3),
                                 (1, 1, 2, 2), 'VALID')


def kernel(image, w1, b1, w2, b2, w3, b3, w4, b4, w5, b5, fc1_w, fc1_b,
           fc2_w, fc2_b):
    B = image.shape[0]

    resized, counts = pl.pallas_call(
        _img_kernel,
        grid=(B,),
        in_specs=[pl.BlockSpec((1, 4, 448, 448), lambda i: (i, 0, 0, 0))],
        out_specs=[
            pl.BlockSpec((1, 3, 112, 112), lambda i: (i, 0, 0, 0)),
            pl.BlockSpec((1, 8, _BINS), lambda i: (i, 0, 0)),
        ],
        out_shape=[
            jax.ShapeDtypeStruct((B, 3, 112, 112), jnp.float32),
            jax.ShapeDtypeStruct((B, 8, _BINS), jnp.float32),
        ],
        compiler_params=pltpu.CompilerParams(
            dimension_semantics=("parallel",)),
    )(image)

    x = _maxpool3s2(jax.nn.relu(_conv(resized, w1, b1, 4, 2)))
    x = _maxpool3s2(jax.nn.relu(_conv(x, w2, b2, 1, 2)))
    x = jax.nn.relu(_conv(x, w3, b3, 1, 1))
    x = jax.nn.relu(_conv(x, w4, b4, 1, 1))
    x = _maxpool3s2(jax.nn.relu(_conv(x, w5, b5, 1, 1)))
    f_sem = x.reshape(B, -1)

    h1 = counts.reshape(B, 2, 2, 2, _BINS) / (224.0 * 224.0)
    h0 = h1.sum(axis=(2, 3)) * 0.25
    out0 = jnp.broadcast_to(h0[:, :, :, None, None], (B, 2, _BINS, 4, 4))
    t = h1.transpose(0, 1, 4, 2, 3)
    out1 = jnp.repeat(jnp.repeat(t, 2, axis=3), 2, axis=4)
    feat = jnp.concatenate([out0, out1, jnp.zeros_like(out0)], axis=2)
    feat = feat.reshape(B, 2, 3 * _BINS * 16)
    f_inten = feat[:, 0]
    f_grad = feat[:, 1]

    out = pl.pallas_call(
        _head_kernel,
        out_shape=jax.ShapeDtypeStruct((B, fc2_w.shape[0]), jnp.float32),
    )(f_sem, f_inten, f_grad, fc1_w.T, fc1_b.reshape(1, -1), fc2_w.T,
      fc2_b.reshape(1, -1))
    return out


# speedup vs baseline: 93.8023x; 1.3541x over previous
import jax
import jax.numpy as jnp
from jax.experimental import pallas as pl
from jax.experimental.pallas import tpu as pltpu

_BINS = 32


def _img_kernel(img_ref, resized_ref, counts_ref):
    x = img_ref[0]
    r = x[0]
    g = 0.5 * (x[1] + x[2])
    b = x[3]
    gray = 0.299 * r + 0.587 * g + 0.114 * b

    ri = jax.lax.broadcasted_iota(jnp.int32, (112, 448), 0)
    ci = jax.lax.broadcasted_iota(jnp.int32, (112, 448), 1)
    P = jnp.where(ci // 4 == ri, 0.25, 0.0).astype(jnp.float32)
    ri2 = jax.lax.broadcasted_iota(jnp.int32, (448, 112), 0)
    ci2 = jax.lax.broadcasted_iota(jnp.int32, (448, 112), 1)
    PT = jnp.where(ri2 // 4 == ci2, 0.25, 0.0).astype(jnp.float32)
    for ch_i, ch in enumerate((r, g, b)):
        t = jnp.dot(P, ch, preferred_element_type=jnp.float32)
        resized_ref[0, ch_i] = jnp.dot(t, PT, preferred_element_type=jnp.float32)

    pr = jnp.concatenate([gray[:1], gray, gray[-1:]], axis=0)
    p = jnp.concatenate([pr[:, :1], pr, pr[:, -1:]], axis=1)
    sv = p[:-2] + 2.0 * p[1:-1] + p[2:]
    gx = (sv[:, 2:] - sv[:, :-2]) * 0.125
    sh = p[:, :-2] + 2.0 * p[:, 1:-1] + p[:, 2:]
    gy = (sh[2:] - sh[:-2]) * 0.125
    mag = jnp.sqrt(gx * gx + gy * gy + 1e-6)

    iidx = jnp.clip((gray * _BINS).astype(jnp.int32), 0, _BINS - 1)
    gidx = jnp.clip((mag * _BINS).astype(jnp.int32), 0, _BINS - 1)

    rows = []
    for idx in (iidx, gidx):
        for base in (0, 224):
            for c in range(_BINS):
                acc = jnp.zeros((8, 448), jnp.float32)
                for i in range(28):
                    sl = idx[base + 8 * i:base + 8 * i + 8]
                    acc = acc + (sl == c).astype(jnp.float32)
                rows.append(jnp.sum(acc, axis=0, keepdims=True))
    r_all = jnp.concatenate(rows, axis=0)
    cw_r = jax.lax.broadcasted_iota(jnp.int32, (448, 2), 0)
    cw_c = jax.lax.broadcasted_iota(jnp.int32, (448, 2), 1)
    cw = jnp.where((cw_r < 224) == (cw_c == 0), 1.0, 0.0).astype(jnp.float32)
    counts_ref[0] = jnp.dot(r_all, cw, preferred_element_type=jnp.float32)


def _head_kernel(sem_ref, inten_ref, grad_ref, w1_ref, b1_ref, w2_ref, b2_ref,
                 out_ref):
    def l2n(v):
        n = jnp.sqrt(jnp.sum(v * v, axis=1, keepdims=True))
        return v / (n + 1e-7)

    s = jax.nn.relu(l2n(sem_ref[:]))
    i = jax.nn.relu(l2n(inten_ref[:]))
    g = jax.nn.relu(l2n(grad_ref[:]))
    h = jnp.concatenate([s, i, g], axis=1)
    h1 = jax.nn.relu(jnp.dot(h, w1_ref[:], preferred_element_type=jnp.float32)
                     + b1_ref[:])
    out_ref[:] = (jnp.dot(h1, w2_ref[:], preferred_element_type=jnp.float32)
                  + b2_ref[:])


def _conv(x, w, b, stride, pad):
    y = jax.lax.conv_general_dilated(x, w, (stride, stride),
                                     [(pad, pad), (pad, pad)],
                                     dimension_numbers=('NCHW', 'OIHW', 'NCHW'))
    return y + b[None, :, None, None]


def _maxpool3s2(x):
    return jax.lax.reduce_window(x, -jnp.inf, jax.lax.max, (1, 1, 3, 3),
                                 (1, 1, 2, 2), 'VALID')


def kernel(image, w1, b1, w2, b2, w3, b3, w4, b4, w5, b5, fc1_w, fc1_b,
           fc2_w, fc2_b):
    B = image.shape[0]

    resized, counts = pl.pallas_call(
        _img_kernel,
        grid=(B,),
        in_specs=[pl.BlockSpec((1, 4, 448, 448), lambda i: (i, 0, 0, 0))],
        out_specs=[
            pl.BlockSpec((1, 3, 112, 112), lambda i: (i, 0, 0, 0)),
            pl.BlockSpec((1, 4 * _BINS, 2), lambda i: (i, 0, 0)),
        ],
        out_shape=[
            jax.ShapeDtypeStruct((B, 3, 112, 112), jnp.float32),
            jax.ShapeDtypeStruct((B, 4 * _BINS, 2), jnp.float32),
        ],
        compiler_params=pltpu.CompilerParams(
            dimension_semantics=("parallel",)),
    )(image)

    x = _maxpool3s2(jax.nn.relu(_conv(resized, w1, b1, 4, 2)))
    x = _maxpool3s2(jax.nn.relu(_conv(x, w2, b2, 1, 2)))
    x = jax.nn.relu(_conv(x, w3, b3, 1, 1))
    x = jax.nn.relu(_conv(x, w4, b4, 1, 1))
    x = _maxpool3s2(jax.nn.relu(_conv(x, w5, b5, 1, 1)))
    f_sem = x.reshape(B, -1)

    h1 = counts.reshape(B, 2, 2, _BINS, 2).transpose(0, 1, 2, 4, 3) \
        / (224.0 * 224.0)
    h0 = h1.sum(axis=(2, 3)) * 0.25
    out0 = jnp.broadcast_to(h0[:, :, :, None, None], (B, 2, _BINS, 4, 4))
    t = h1.transpose(0, 1, 4, 2, 3)
    out1 = jnp.repeat(jnp.repeat(t, 2, axis=3), 2, axis=4)
    feat = jnp.concatenate([out0, out1, jnp.zeros_like(out0)], axis=2)
    feat = feat.reshape(B, 2, 3 * _BINS * 16)
    f_inten = feat[:, 0]
    f_grad = feat[:, 1]

    out = pl.pallas_call(
        _head_kernel,
        out_shape=jax.ShapeDtypeStruct((B, fc2_w.shape[0]), jnp.float32),
    )(f_sem, f_inten, f_grad, fc1_w.T, fc1_b.reshape(1, -1), fc2_w.T,
      fc2_b.reshape(1, -1))
    return out
